# baseline (speedup 1.0000x reference)
import jax
import jax.numpy as jnp
from jax import lax
from jax.experimental import pallas as pl
from jax.experimental.pallas import tpu as pltpu
from jax.experimental.pallas import tpu_sc as plsc

T, D, H, E = 2048, 768, 3072, 8
BTS = 128
PADTOT = 4096 + E * BTS
NTS = PADTOT // BTS
CH = 512
NW = 32
TPW = T // NW


def _router_body(x_ref, wr_ref, g0_ref, g1_ref, p0_ref, p1_ref,
                 eid_ref, ntu_ref, chg_ref, slot_ref, pref_ref, lb_ref):
    x = x_ref[...]
    wr = wr_ref[...]
    logits = lax.dot_general(
        x, wr, (((1,), (1,)), ((), ())), preferred_element_type=jnp.float32)
    m = jnp.max(logits, axis=1, keepdims=True)
    ex = jnp.exp(logits - m)
    probs = ex / jnp.sum(ex, axis=1, keepdims=True)
    lane = lax.broadcasted_iota(jnp.int32, (T, E), 1)
    p1 = jnp.max(probs, axis=1, keepdims=True)
    a1 = jnp.min(jnp.where(probs >= p1, lane, E), axis=1, keepdims=True)
    m1 = lane == a1
    pm = jnp.where(m1, -1.0, probs)
    p2 = jnp.max(pm, axis=1, keepdims=True)
    a2 = jnp.min(jnp.where(pm >= p2, lane, E), axis=1, keepdims=True)
    m2 = lane == a2
    denom = p1 + p2 + 1e-9
    g0_ref[...] = jnp.broadcast_to(p1 / denom, (T, 16))
    g1_ref[...] = jnp.broadcast_to(p2 / denom, (T, 16))

    imp = jnp.sum(probs, axis=0, keepdims=True)
    load = jnp.sum(m1.astype(jnp.float32) + m2.astype(jnp.float32),
                   axis=0, keepdims=True)
    impn = imp / (jnp.sum(imp) + 1e-9)
    loadn = load / (jnp.sum(load) + 1e-9)
    lb_ref[...] = jnp.sum(impn * loadn, axis=1, keepdims=True) * E

    oh0 = m1.astype(jnp.float32)
    oh1 = m2.astype(jnp.float32)
    ri = lax.broadcasted_iota(jnp.int32, (CH, CH), 0)
    ci = lax.broadcasted_iota(jnp.int32, (CH, CH), 1)
    ls = (ci < ri).astype(jnp.float32)

    def _excl_cumsum(oh):
        carry = jnp.zeros((1, E), jnp.float32)
        parts = []
        for c in range(T // CH):
            blk = oh[c * CH:(c + 1) * CH, :]
            parts.append(lax.dot_general(
                ls, blk, (((1,), (0,)), ((), ())),
                preferred_element_type=jnp.float32) + carry)
            carry = carry + jnp.sum(blk, axis=0, keepdims=True)
        return jnp.concatenate(parts, axis=0), carry

    rank0, count0 = _excl_cumsum(oh0)
    rank1, count1 = _excl_cumsum(oh1)
    rank1 = rank1 + count0
    count = count0 + count1
    padded = jnp.floor((count + (BTS - 1)) * (1.0 / BTS)) * BTS
    er = lax.broadcasted_iota(jnp.int32, (E, E), 0)
    ec = lax.broadcasted_iota(jnp.int32, (E, E), 1)
    mstrict = (er < ec).astype(jnp.float32)
    offs = lax.dot_general(padded, mstrict, (((1,), (0,)), ((), ())),
                           preferred_element_type=jnp.float32)
    p0 = jnp.sum(jnp.where(m1, offs + rank0, 0.0), axis=1, keepdims=True)
    p1i = jnp.sum(jnp.where(m2, offs + rank1, 0.0), axis=1, keepdims=True)
    p0_ref[...] = p0.astype(jnp.int32).reshape(T)
    p1_ref[...] = p1i.astype(jnp.int32).reshape(T)

    ends = offs + padded
    jstart = (lax.broadcasted_iota(jnp.int32, (NTS, 1), 0) * BTS
              ).astype(jnp.float32)
    total = jnp.sum(padded, axis=1, keepdims=True)
    cnt = jnp.sum((jnp.broadcast_to(ends, (NTS, E)) <=
                   jnp.broadcast_to(jstart, (NTS, E))).astype(jnp.int32),
                  axis=1, keepdims=True)
    laste = jnp.sum((ends <= total - 1.0).astype(jnp.int32),
                    axis=1, keepdims=True)
    eid = jnp.where(jstart < total, cnt, laste)
    eid_ref[...] = eid.reshape(NTS)
    ntu_ref[...] = (total * (1.0 / BTS)).astype(jnp.int32).reshape(1)

    prev = jnp.concatenate(
        [jnp.full((1, 1), -1, jnp.int32), eid[:-1, :]], axis=0)
    chg = (eid != prev).astype(jnp.int32)
    ti = lax.broadcasted_iota(jnp.int32, (NTS, NTS), 0)
    tj = lax.broadcasted_iota(jnp.int32, (NTS, NTS), 1)
    lsi = (tj <= ti).astype(jnp.float32)
    csum = lax.dot_general(lsi, chg.astype(jnp.float32),
                           (((1,), (0,)), ((), ())),
                           preferred_element_type=jnp.float32)
    slot = lax.rem(csum.astype(jnp.int32) - 1, 2)
    lane2 = lax.broadcasted_iota(jnp.int32, (NTS, E), 1)
    present = jnp.broadcast_to(count > 0.0, (NTS, E))
    cand = jnp.where(present & (lane2 > jnp.broadcast_to(eid, (NTS, E))),
                     lane2, E)
    pref = jnp.min(cand, axis=1, keepdims=True)
    chg_ref[...] = chg.reshape(NTS)
    slot_ref[...] = slot.reshape(NTS)
    pref_ref[...] = pref.reshape(NTS)


def _router(x2, Wr):
    return pl.pallas_call(
        _router_body,
        out_shape=(
            jax.ShapeDtypeStruct((T, 16), jnp.float32),
            jax.ShapeDtypeStruct((T, 16), jnp.float32),
            jax.ShapeDtypeStruct((T,), jnp.int32),
            jax.ShapeDtypeStruct((T,), jnp.int32),
            jax.ShapeDtypeStruct((NTS,), jnp.int32),
            jax.ShapeDtypeStruct((1,), jnp.int32),
            jax.ShapeDtypeStruct((NTS,), jnp.int32),
            jax.ShapeDtypeStruct((NTS,), jnp.int32),
            jax.ShapeDtypeStruct((NTS,), jnp.int32),
            jax.ShapeDtypeStruct((1, 1), jnp.float32),
        ),
    )(x2, Wr)


def _sc_mesh():
    return plsc.VectorSubcoreMesh(core_axis_name="c", subcore_axis_name="s")


def _dispatch_body(x_hbm, p0_hbm, p1_hbm, xs_hbm, idx0_v, idx1_v, xbuf_v,
                   sem0, sem1):
    wid = lax.axis_index("s") * 2 + lax.axis_index("c")
    base = wid * TPW
    pltpu.sync_copy(p0_hbm.at[pl.ds(base, TPW)], idx0_v)
    pltpu.sync_copy(p1_hbm.at[pl.ds(base, TPW)], idx1_v)
    pltpu.sync_copy(x_hbm.at[pl.ds(base, TPW)], xbuf_v)
    c0 = pltpu.async_copy(xbuf_v, xs_hbm.at[idx0_v], sem0)
    c1 = pltpu.async_copy(xbuf_v, xs_hbm.at[idx1_v], sem1)
    c0.wait()
    c1.wait()


def _sc_dispatch(x2, p0, p1):
    return pl.kernel(
        _dispatch_body,
        out_type=jax.ShapeDtypeStruct((PADTOT, D), jnp.float32),
        mesh=_sc_mesh(),
        scratch_types=[
            pltpu.VMEM((TPW,), jnp.int32),
            pltpu.VMEM((TPW,), jnp.int32),
            pltpu.VMEM((TPW, D), jnp.float32),
            pltpu.SemaphoreType.DMA,
            pltpu.SemaphoreType.DMA,
        ],
    )(x2, p0, p1)


def _w_copies(w1_hbm, w2_hbm, w1_v, w2_v, sem1, sem2, e, s):
    c1 = pltpu.make_async_copy(w1_hbm.at[e], w1_v.at[s], sem1)
    c2 = pltpu.make_async_copy(w2_hbm.at[e], w2_v.at[s], sem2)
    return c1, c2


def _gffn_body(eid_ref, ntu_ref, chg_ref, slot_ref, pref_ref,
               xs_ref, w1_hbm, w2_hbm, ys_ref, w1_v, w2_v,
               sem1a, sem2a, sem1b, sem2b):
    j = pl.program_id(0)
    s = slot_ref[j]
    nxt = pref_ref[j]

    @pl.when(chg_ref[j] == 1)
    def _():
        @pl.when(j == 0)
        def _():
            c1, c2 = _w_copies(w1_hbm, w2_hbm, w1_v, w2_v, sem1a, sem2a,
                               eid_ref[0], 0)
            c1.start()
            c2.start()

        @pl.when(s == 0)
        def _():
            c1, c2 = _w_copies(w1_hbm, w2_hbm, w1_v, w2_v, sem1a, sem2a,
                               eid_ref[j], 0)
            c1.wait()
            c2.wait()

        @pl.when(s == 1)
        def _():
            c1, c2 = _w_copies(w1_hbm, w2_hbm, w1_v, w2_v, sem1b, sem2b,
                               eid_ref[j], 1)
            c1.wait()
            c2.wait()

        @pl.when((nxt < E) & (s == 0))
        def _():
            c1, c2 = _w_copies(w1_hbm, w2_hbm, w1_v, w2_v, sem1b, sem2b,
                               nxt, 1)
            c1.start()
            c2.start()

        @pl.when((nxt < E) & (s == 1))
        def _():
            c1, c2 = _w_copies(w1_hbm, w2_hbm, w1_v, w2_v, sem1a, sem2a,
                               nxt, 0)
            c1.start()
            c2.start()

    def _compute(slot_static):
        xb = xs_ref[...].astype(jnp.bfloat16)
        hpre = lax.dot_general(
            xb, w1_v[slot_static], (((1,), (1,)), ((), ())),
            preferred_element_type=jnp.float32)
        hact = (hpre * 0.5 * (1.0 + lax.erf(hpre * 0.7071067811865476))
                ).astype(jnp.bfloat16)
        ys_ref[...] = lax.dot_general(
            hact, w2_v[slot_static], (((1,), (1,)), ((), ())),
            preferred_element_type=jnp.float32)

    @pl.when((j < ntu_ref[0]) & (s == 0))
    def _():
        _compute(0)

    @pl.when((j < ntu_ref[0]) & (s == 1))
    def _():
        _compute(1)


def _gffn(eid, ntu, chg, slot, pref, xs, W1b, W2b):
    grid_spec = pltpu.PrefetchScalarGridSpec(
        num_scalar_prefetch=5,
        grid=(NTS,),
        in_specs=[
            pl.BlockSpec((BTS, D), lambda j, *_: (j, 0)),
            pl.BlockSpec(memory_space=pl.ANY),
            pl.BlockSpec(memory_space=pl.ANY),
        ],
        out_specs=pl.BlockSpec((BTS, D), lambda j, *_: (j, 0)),
        scratch_shapes=[
            pltpu.VMEM((2, H, D), jnp.bfloat16),
            pltpu.VMEM((2, D, H), jnp.bfloat16),
            pltpu.SemaphoreType.DMA,
            pltpu.SemaphoreType.DMA,
            pltpu.SemaphoreType.DMA,
            pltpu.SemaphoreType.DMA,
        ],
    )
    return pl.pallas_call(
        _gffn_body,
        grid_spec=grid_spec,
        out_shape=jax.ShapeDtypeStruct((PADTOT, D), jnp.float32),
    )(eid, ntu, chg, slot, pref, xs, W1b, W2b)


def _combine_body(ys_hbm, p0_hbm, p1_hbm, g0_hbm, g1_hbm, out_hbm,
                  idx0_v, idx1_v, g0_v, g1_v, y0_v, y1_v, sem0, sem1):
    wid = lax.axis_index("s") * 2 + lax.axis_index("c")
    base = wid * TPW
    pltpu.sync_copy(p0_hbm.at[pl.ds(base, TPW)], idx0_v)
    pltpu.sync_copy(p1_hbm.at[pl.ds(base, TPW)], idx1_v)
    pltpu.sync_copy(g0_hbm.at[pl.ds(base, TPW)], g0_v)
    pltpu.sync_copy(g1_hbm.at[pl.ds(base, TPW)], g1_v)
    c0 = pltpu.async_copy(ys_hbm.at[idx0_v], y0_v, sem0)
    c1 = pltpu.async_copy(ys_hbm.at[idx1_v], y1_v, sem1)
    c0.wait()
    c1.wait()

    def _row(i, acc):
        g0 = g0_v[i]
        g1 = g1_v[i]
        for q in range(D // 16):
            sl = pl.ds(q * 16, 16)
            y0_v[i, sl] = y0_v[i, sl] * g0 + y1_v[i, sl] * g1
        return acc

    lax.fori_loop(0, TPW, _row, 0)
    pltpu.sync_copy(y0_v, out_hbm.at[pl.ds(base, TPW)])


def _sc_combine(ys, p0, p1, g0, g1):
    return pl.kernel(
        _combine_body,
        out_type=jax.ShapeDtypeStruct((T, D), jnp.float32),
        mesh=_sc_mesh(),
        scratch_types=[
            pltpu.VMEM((TPW,), jnp.int32),
            pltpu.VMEM((TPW,), jnp.int32),
            pltpu.VMEM((TPW, 16), jnp.float32),
            pltpu.VMEM((TPW, 16), jnp.float32),
            pltpu.VMEM((TPW, D), jnp.float32),
            pltpu.VMEM((TPW, D), jnp.float32),
            pltpu.SemaphoreType.DMA,
            pltpu.SemaphoreType.DMA,
        ],
    )(ys, p0, p1, g0, g1)


def kernel(x, Wr, W1, W2):
    b, t, d = x.shape
    x2 = x.reshape(t, d)
    g0, g1, p0, p1, eid, ntu, chg, slot, pref, lb = _router(x2, Wr)
    xs = _sc_dispatch(x2, p0, p1)
    ys = _gffn(eid, ntu, chg, slot, pref, xs,
               W1.astype(jnp.bfloat16), W2.astype(jnp.bfloat16))
    out = _sc_combine(ys, p0, p1, g0, g1)
    return out.reshape(b, t, d), lb.reshape(())


# speedup vs baseline: 1.2876x; 1.2876x over previous
import jax
import jax.numpy as jnp
from jax import lax
from jax.experimental import pallas as pl
from jax.experimental.pallas import tpu as pltpu
from jax.experimental.pallas import tpu_sc as plsc

T, D, H, E = 2048, 768, 3072, 8
BTS = 512
PADTOT = 4096 + E * BTS
NTS = PADTOT // BTS
CH = 512
NW = 32
TPW = T // NW


def _router_body(x_ref, wr_ref, g0_ref, g1_ref, p0_ref, p1_ref,
                 eid_ref, ntu_ref, chg_ref, slot_ref, pref_ref, lb_ref):
    x = x_ref[...]
    wr = wr_ref[...]
    logits = lax.dot_general(
        x, wr, (((1,), (1,)), ((), ())), preferred_element_type=jnp.float32)
    m = jnp.max(logits, axis=1, keepdims=True)
    ex = jnp.exp(logits - m)
    probs = ex / jnp.sum(ex, axis=1, keepdims=True)
    lane = lax.broadcasted_iota(jnp.int32, (T, E), 1)
    p1 = jnp.max(probs, axis=1, keepdims=True)
    a1 = jnp.min(jnp.where(probs >= p1, lane, E), axis=1, keepdims=True)
    m1 = lane == a1
    pm = jnp.where(m1, -1.0, probs)
    p2 = jnp.max(pm, axis=1, keepdims=True)
    a2 = jnp.min(jnp.where(pm >= p2, lane, E), axis=1, keepdims=True)
    m2 = lane == a2
    denom = p1 + p2 + 1e-9
    g0_ref[...] = jnp.broadcast_to(p1 / denom, (T, 16))
    g1_ref[...] = jnp.broadcast_to(p2 / denom, (T, 16))

    imp = jnp.sum(probs, axis=0, keepdims=True)
    load = jnp.sum(m1.astype(jnp.float32) + m2.astype(jnp.float32),
                   axis=0, keepdims=True)
    impn = imp / (jnp.sum(imp) + 1e-9)
    loadn = load / (jnp.sum(load) + 1e-9)
    lb_ref[...] = jnp.sum(impn * loadn, axis=1, keepdims=True) * E

    oh0 = m1.astype(jnp.float32)
    oh1 = m2.astype(jnp.float32)
    ri = lax.broadcasted_iota(jnp.int32, (CH, CH), 0)
    ci = lax.broadcasted_iota(jnp.int32, (CH, CH), 1)
    ls = (ci < ri).astype(jnp.float32)

    def _excl_cumsum(oh):
        carry = jnp.zeros((1, E), jnp.float32)
        parts = []
        for c in range(T // CH):
            blk = oh[c * CH:(c + 1) * CH, :]
            parts.append(lax.dot_general(
                ls, blk, (((1,), (0,)), ((), ())),
                preferred_element_type=jnp.float32) + carry)
            carry = carry + jnp.sum(blk, axis=0, keepdims=True)
        return jnp.concatenate(parts, axis=0), carry

    rank0, count0 = _excl_cumsum(oh0)
    rank1, count1 = _excl_cumsum(oh1)
    rank1 = rank1 + count0
    count = count0 + count1
    padded = jnp.floor((count + (BTS - 1)) * (1.0 / BTS)) * BTS
    er = lax.broadcasted_iota(jnp.int32, (E, E), 0)
    ec = lax.broadcasted_iota(jnp.int32, (E, E), 1)
    mstrict = (er < ec).astype(jnp.float32)
    offs = lax.dot_general(padded, mstrict, (((1,), (0,)), ((), ())),
                           preferred_element_type=jnp.float32)
    p0 = jnp.sum(jnp.where(m1, offs + rank0, 0.0), axis=1, keepdims=True)
    p1i = jnp.sum(jnp.where(m2, offs + rank1, 0.0), axis=1, keepdims=True)
    p0_ref[...] = p0.astype(jnp.int32).reshape(T)
    p1_ref[...] = p1i.astype(jnp.int32).reshape(T)

    ends = offs + padded
    jstart = (lax.broadcasted_iota(jnp.int32, (NTS, 1), 0) * BTS
              ).astype(jnp.float32)
    total = jnp.sum(padded, axis=1, keepdims=True)
    cnt = jnp.sum((jnp.broadcast_to(ends, (NTS, E)) <=
                   jnp.broadcast_to(jstart, (NTS, E))).astype(jnp.int32),
                  axis=1, keepdims=True)
    laste = jnp.sum((ends <= total - 1.0).astype(jnp.int32),
                    axis=1, keepdims=True)
    eid = jnp.where(jstart < total, cnt, laste)
    eid_ref[...] = eid.reshape(NTS)
    ntu_ref[...] = (total * (1.0 / BTS)).astype(jnp.int32).reshape(1)

    prev = jnp.concatenate(
        [jnp.full((1, 1), -1, jnp.int32), eid[:-1, :]], axis=0)
    chg = (eid != prev).astype(jnp.int32)
    ti = lax.broadcasted_iota(jnp.int32, (NTS, NTS), 0)
    tj = lax.broadcasted_iota(jnp.int32, (NTS, NTS), 1)
    lsi = (tj <= ti).astype(jnp.float32)
    csum = lax.dot_general(lsi, chg.astype(jnp.float32),
                           (((1,), (0,)), ((), ())),
                           preferred_element_type=jnp.float32)
    slot = lax.rem(csum.astype(jnp.int32) - 1, 2)
    lane2 = lax.broadcasted_iota(jnp.int32, (NTS, E), 1)
    present = jnp.broadcast_to(count > 0.0, (NTS, E))
    cand = jnp.where(present & (lane2 > jnp.broadcast_to(eid, (NTS, E))),
                     lane2, E)
    pref = jnp.min(cand, axis=1, keepdims=True)
    chg_ref[...] = chg.reshape(NTS)
    slot_ref[...] = slot.reshape(NTS)
    pref_ref[...] = pref.reshape(NTS)


def _router(x2, Wr):
    return pl.pallas_call(
        _router_body,
        out_shape=(
            jax.ShapeDtypeStruct((T, 16), jnp.float32),
            jax.ShapeDtypeStruct((T, 16), jnp.float32),
            jax.ShapeDtypeStruct((T,), jnp.int32),
            jax.ShapeDtypeStruct((T,), jnp.int32),
            jax.ShapeDtypeStruct((NTS,), jnp.int32),
            jax.ShapeDtypeStruct((1,), jnp.int32),
            jax.ShapeDtypeStruct((NTS,), jnp.int32),
            jax.ShapeDtypeStruct((NTS,), jnp.int32),
            jax.ShapeDtypeStruct((NTS,), jnp.int32),
            jax.ShapeDtypeStruct((1, 1), jnp.float32),
        ),
    )(x2, Wr)


def _sc_mesh():
    return plsc.VectorSubcoreMesh(core_axis_name="c", subcore_axis_name="s")


def _dispatch_body(x_hbm, p0_hbm, p1_hbm, xs_hbm, idx0_v, idx1_v, xbuf_v,
                   sem0, sem1):
    wid = lax.axis_index("s") * 2 + lax.axis_index("c")
    base = wid * TPW
    pltpu.sync_copy(p0_hbm.at[pl.ds(base, TPW)], idx0_v)
    pltpu.sync_copy(p1_hbm.at[pl.ds(base, TPW)], idx1_v)
    pltpu.sync_copy(x_hbm.at[pl.ds(base, TPW)], xbuf_v)
    c0 = pltpu.async_copy(xbuf_v, xs_hbm.at[idx0_v], sem0)
    c1 = pltpu.async_copy(xbuf_v, xs_hbm.at[idx1_v], sem1)
    c0.wait()
    c1.wait()


def _sc_dispatch(x2, p0, p1):
    return pl.kernel(
        _dispatch_body,
        out_type=jax.ShapeDtypeStruct((PADTOT, D), jnp.float32),
        mesh=_sc_mesh(),
        scratch_types=[
            pltpu.VMEM((TPW,), jnp.int32),
            pltpu.VMEM((TPW,), jnp.int32),
            pltpu.VMEM((TPW, D), jnp.float32),
            pltpu.SemaphoreType.DMA,
            pltpu.SemaphoreType.DMA,
        ],
    )(x2, p0, p1)


def _w_copies(w1_hbm, w2_hbm, w1_v, w2_v, sem1, sem2, e, s):
    c1 = pltpu.make_async_copy(w1_hbm.at[e], w1_v.at[s], sem1)
    c2 = pltpu.make_async_copy(w2_hbm.at[e], w2_v.at[s], sem2)
    return c1, c2


def _gffn_body(eid_ref, ntu_ref, chg_ref, slot_ref, pref_ref,
               xs_ref, w1_hbm, w2_hbm, ys_ref, w1_v, w2_v,
               sem1a, sem2a, sem1b, sem2b):
    j = pl.program_id(0)
    s = slot_ref[j]
    nxt = pref_ref[j]

    @pl.when(chg_ref[j] == 1)
    def _():
        @pl.when(j == 0)
        def _():
            c1, c2 = _w_copies(w1_hbm, w2_hbm, w1_v, w2_v, sem1a, sem2a,
                               eid_ref[0], 0)
            c1.start()
            c2.start()

        @pl.when(s == 0)
        def _():
            c1, c2 = _w_copies(w1_hbm, w2_hbm, w1_v, w2_v, sem1a, sem2a,
                               eid_ref[j], 0)
            c1.wait()
            c2.wait()

        @pl.when(s == 1)
        def _():
            c1, c2 = _w_copies(w1_hbm, w2_hbm, w1_v, w2_v, sem1b, sem2b,
                               eid_ref[j], 1)
            c1.wait()
            c2.wait()

        @pl.when((nxt < E) & (s == 0))
        def _():
            c1, c2 = _w_copies(w1_hbm, w2_hbm, w1_v, w2_v, sem1b, sem2b,
                               nxt, 1)
            c1.start()
            c2.start()

        @pl.when((nxt < E) & (s == 1))
        def _():
            c1, c2 = _w_copies(w1_hbm, w2_hbm, w1_v, w2_v, sem1a, sem2a,
                               nxt, 0)
            c1.start()
            c2.start()

    def _compute(slot_static):
        xb = xs_ref[...].astype(jnp.bfloat16)
        hpre = lax.dot_general(
            xb, w1_v[slot_static], (((1,), (1,)), ((), ())),
            preferred_element_type=jnp.float32)
        hact = (hpre * 0.5 * (1.0 + lax.erf(hpre * 0.7071067811865476))
                ).astype(jnp.bfloat16)
        ys_ref[...] = lax.dot_general(
            hact, w2_v[slot_static], (((1,), (1,)), ((), ())),
            preferred_element_type=jnp.float32)

    @pl.when((j < ntu_ref[0]) & (s == 0))
    def _():
        _compute(0)

    @pl.when((j < ntu_ref[0]) & (s == 1))
    def _():
        _compute(1)


def _gffn(eid, ntu, chg, slot, pref, xs, W1b, W2b):
    grid_spec = pltpu.PrefetchScalarGridSpec(
        num_scalar_prefetch=5,
        grid=(NTS,),
        in_specs=[
            pl.BlockSpec((BTS, D), lambda j, *_: (j, 0)),
            pl.BlockSpec(memory_space=pl.ANY),
            pl.BlockSpec(memory_space=pl.ANY),
        ],
        out_specs=pl.BlockSpec((BTS, D), lambda j, *_: (j, 0)),
        scratch_shapes=[
            pltpu.VMEM((2, H, D), jnp.bfloat16),
            pltpu.VMEM((2, D, H), jnp.bfloat16),
            pltpu.SemaphoreType.DMA,
            pltpu.SemaphoreType.DMA,
            pltpu.SemaphoreType.DMA,
            pltpu.SemaphoreType.DMA,
        ],
    )
    return pl.pallas_call(
        _gffn_body,
        grid_spec=grid_spec,
        out_shape=jax.ShapeDtypeStruct((PADTOT, D), jnp.float32),
    )(eid, ntu, chg, slot, pref, xs, W1b, W2b)


def _combine_body(ys_hbm, p0_hbm, p1_hbm, g0_hbm, g1_hbm, out_hbm,
                  idx0_v, idx1_v, g0_v, g1_v, y0_v, y1_v, sem0, sem1):
    wid = lax.axis_index("s") * 2 + lax.axis_index("c")
    base = wid * TPW
    pltpu.sync_copy(p0_hbm.at[pl.ds(base, TPW)], idx0_v)
    pltpu.sync_copy(p1_hbm.at[pl.ds(base, TPW)], idx1_v)
    pltpu.sync_copy(g0_hbm.at[pl.ds(base, TPW)], g0_v)
    pltpu.sync_copy(g1_hbm.at[pl.ds(base, TPW)], g1_v)
    c0 = pltpu.async_copy(ys_hbm.at[idx0_v], y0_v, sem0)
    c1 = pltpu.async_copy(ys_hbm.at[idx1_v], y1_v, sem1)
    c0.wait()
    c1.wait()

    def _row(i, acc):
        g0 = g0_v[i]
        g1 = g1_v[i]
        for q in range(D // 16):
            sl = pl.ds(q * 16, 16)
            y0_v[i, sl] = y0_v[i, sl] * g0 + y1_v[i, sl] * g1
        return acc

    lax.fori_loop(0, TPW, _row, 0)
    pltpu.sync_copy(y0_v, out_hbm.at[pl.ds(base, TPW)])


def _sc_combine(ys, p0, p1, g0, g1):
    return pl.kernel(
        _combine_body,
        out_type=jax.ShapeDtypeStruct((T, D), jnp.float32),
        mesh=_sc_mesh(),
        scratch_types=[
            pltpu.VMEM((TPW,), jnp.int32),
            pltpu.VMEM((TPW,), jnp.int32),
            pltpu.VMEM((TPW, 16), jnp.float32),
            pltpu.VMEM((TPW, 16), jnp.float32),
            pltpu.VMEM((TPW, D), jnp.float32),
            pltpu.VMEM((TPW, D), jnp.float32),
            pltpu.SemaphoreType.DMA,
            pltpu.SemaphoreType.DMA,
        ],
    )(ys, p0, p1, g0, g1)


def kernel(x, Wr, W1, W2):
    b, t, d = x.shape
    x2 = x.reshape(t, d)
    g0, g1, p0, p1, eid, ntu, chg, slot, pref, lb = _router(x2, Wr)
    xs = _sc_dispatch(x2, p0, p1)
    ys = _gffn(eid, ntu, chg, slot, pref, xs,
               W1.astype(jnp.bfloat16), W2.astype(jnp.bfloat16))
    out = _sc_combine(ys, p0, p1, g0, g1)
    return out.reshape(b, t, d), lb.reshape(())


# speedup vs baseline: 1.3292x; 1.0323x over previous
import jax
import jax.numpy as jnp
from jax import lax
from jax.experimental import pallas as pl
from jax.experimental.pallas import tpu as pltpu
from jax.experimental.pallas import tpu_sc as plsc

T, D, H, E = 2048, 768, 3072, 8
BTS = 256
PADTOT = 4096 + E * BTS
NTS = PADTOT // BTS
CH = 512
NW = 32
TPW = T // NW


def _router_body(x_ref, wr_ref, g0_ref, g1_ref, p0_ref, p1_ref,
                 eid_ref, ntu_ref, chg_ref, slot_ref, pref_ref, lb_ref):
    x = x_ref[...]
    wr = wr_ref[...]
    logits = lax.dot_general(
        x, wr, (((1,), (1,)), ((), ())), preferred_element_type=jnp.float32)
    m = jnp.max(logits, axis=1, keepdims=True)
    ex = jnp.exp(logits - m)
    probs = ex / jnp.sum(ex, axis=1, keepdims=True)
    lane = lax.broadcasted_iota(jnp.int32, (T, E), 1)
    p1 = jnp.max(probs, axis=1, keepdims=True)
    a1 = jnp.min(jnp.where(probs >= p1, lane, E), axis=1, keepdims=True)
    m1 = lane == a1
    pm = jnp.where(m1, -1.0, probs)
    p2 = jnp.max(pm, axis=1, keepdims=True)
    a2 = jnp.min(jnp.where(pm >= p2, lane, E), axis=1, keepdims=True)
    m2 = lane == a2
    denom = p1 + p2 + 1e-9
    g0_ref[...] = jnp.broadcast_to(p1 / denom, (T, 16))
    g1_ref[...] = jnp.broadcast_to(p2 / denom, (T, 16))

    imp = jnp.sum(probs, axis=0, keepdims=True)
    load = jnp.sum(m1.astype(jnp.float32) + m2.astype(jnp.float32),
                   axis=0, keepdims=True)
    impn = imp / (jnp.sum(imp) + 1e-9)
    loadn = load / (jnp.sum(load) + 1e-9)
    lb_ref[...] = jnp.sum(impn * loadn, axis=1, keepdims=True) * E

    oh0 = m1.astype(jnp.float32)
    oh1 = m2.astype(jnp.float32)
    ri = lax.broadcasted_iota(jnp.int32, (CH, CH), 0)
    ci = lax.broadcasted_iota(jnp.int32, (CH, CH), 1)
    ls = (ci < ri).astype(jnp.float32)

    def _excl_cumsum(oh):
        carry = jnp.zeros((1, E), jnp.float32)
        parts = []
        for c in range(T // CH):
            blk = oh[c * CH:(c + 1) * CH, :]
            parts.append(lax.dot_general(
                ls, blk, (((1,), (0,)), ((), ())),
                preferred_element_type=jnp.float32) + carry)
            carry = carry + jnp.sum(blk, axis=0, keepdims=True)
        return jnp.concatenate(parts, axis=0), carry

    rank0, count0 = _excl_cumsum(oh0)
    rank1, count1 = _excl_cumsum(oh1)
    rank1 = rank1 + count0
    count = count0 + count1
    padded = jnp.floor((count + (BTS - 1)) * (1.0 / BTS)) * BTS
    er = lax.broadcasted_iota(jnp.int32, (E, E), 0)
    ec = lax.broadcasted_iota(jnp.int32, (E, E), 1)
    mstrict = (er < ec).astype(jnp.float32)
    offs = lax.dot_general(padded, mstrict, (((1,), (0,)), ((), ())),
                           preferred_element_type=jnp.float32)
    p0 = jnp.sum(jnp.where(m1, offs + rank0, 0.0), axis=1, keepdims=True)
    p1i = jnp.sum(jnp.where(m2, offs + rank1, 0.0), axis=1, keepdims=True)
    p0_ref[...] = p0.astype(jnp.int32).reshape(T)
    p1_ref[...] = p1i.astype(jnp.int32).reshape(T)

    ends = offs + padded
    jstart = (lax.broadcasted_iota(jnp.int32, (NTS, 1), 0) * BTS
              ).astype(jnp.float32)
    total = jnp.sum(padded, axis=1, keepdims=True)
    cnt = jnp.sum((jnp.broadcast_to(ends, (NTS, E)) <=
                   jnp.broadcast_to(jstart, (NTS, E))).astype(jnp.int32),
                  axis=1, keepdims=True)
    laste = jnp.sum((ends <= total - 1.0).astype(jnp.int32),
                    axis=1, keepdims=True)
    eid = jnp.where(jstart < total, cnt, laste)
    eid_ref[...] = eid.reshape(NTS)
    ntu_ref[...] = (total * (1.0 / BTS)).astype(jnp.int32).reshape(1)

    prev = jnp.concatenate(
        [jnp.full((1, 1), -1, jnp.int32), eid[:-1, :]], axis=0)
    chg = (eid != prev).astype(jnp.int32)
    ti = lax.broadcasted_iota(jnp.int32, (NTS, NTS), 0)
    tj = lax.broadcasted_iota(jnp.int32, (NTS, NTS), 1)
    lsi = (tj <= ti).astype(jnp.float32)
    csum = lax.dot_general(lsi, chg.astype(jnp.float32),
                           (((1,), (0,)), ((), ())),
                           preferred_element_type=jnp.float32)
    slot = lax.rem(csum.astype(jnp.int32) - 1, 2)
    lane2 = lax.broadcasted_iota(jnp.int32, (NTS, E), 1)
    present = jnp.broadcast_to(count > 0.0, (NTS, E))
    cand = jnp.where(present & (lane2 > jnp.broadcast_to(eid, (NTS, E))),
                     lane2, E)
    pref = jnp.min(cand, axis=1, keepdims=True)
    chg_ref[...] = chg.reshape(NTS)
    slot_ref[...] = slot.reshape(NTS)
    pref_ref[...] = pref.reshape(NTS)


def _router(x2, Wr):
    return pl.pallas_call(
        _router_body,
        out_shape=(
            jax.ShapeDtypeStruct((T, 16), jnp.float32),
            jax.ShapeDtypeStruct((T, 16), jnp.float32),
            jax.ShapeDtypeStruct((T,), jnp.int32),
            jax.ShapeDtypeStruct((T,), jnp.int32),
            jax.ShapeDtypeStruct((NTS,), jnp.int32),
            jax.ShapeDtypeStruct((1,), jnp.int32),
            jax.ShapeDtypeStruct((NTS,), jnp.int32),
            jax.ShapeDtypeStruct((NTS,), jnp.int32),
            jax.ShapeDtypeStruct((NTS,), jnp.int32),
            jax.ShapeDtypeStruct((1, 1), jnp.float32),
        ),
    )(x2, Wr)


def _sc_mesh():
    return plsc.VectorSubcoreMesh(core_axis_name="c", subcore_axis_name="s")


def _dispatch_body(x_hbm, p0_hbm, p1_hbm, xs_hbm, idx0_v, idx1_v, xbuf_v,
                   sem0, sem1):
    wid = lax.axis_index("s") * 2 + lax.axis_index("c")
    base = wid * TPW
    pltpu.sync_copy(p0_hbm.at[pl.ds(base, TPW)], idx0_v)
    pltpu.sync_copy(p1_hbm.at[pl.ds(base, TPW)], idx1_v)
    pltpu.sync_copy(x_hbm.at[pl.ds(base, TPW)], xbuf_v)
    c0 = pltpu.async_copy(xbuf_v, xs_hbm.at[idx0_v], sem0)
    c1 = pltpu.async_copy(xbuf_v, xs_hbm.at[idx1_v], sem1)
    c0.wait()
    c1.wait()


def _sc_dispatch(x2, p0, p1):
    return pl.kernel(
        _dispatch_body,
        out_type=jax.ShapeDtypeStruct((PADTOT, D), jnp.float32),
        mesh=_sc_mesh(),
        scratch_types=[
            pltpu.VMEM((TPW,), jnp.int32),
            pltpu.VMEM((TPW,), jnp.int32),
            pltpu.VMEM((TPW, D), jnp.float32),
            pltpu.SemaphoreType.DMA,
            pltpu.SemaphoreType.DMA,
        ],
    )(x2, p0, p1)


def _w_copies(w1_hbm, w2_hbm, w1_v, w2_v, sem1, sem2, e, s):
    c1 = pltpu.make_async_copy(w1_hbm.at[e], w1_v.at[s], sem1)
    c2 = pltpu.make_async_copy(w2_hbm.at[e], w2_v.at[s], sem2)
    return c1, c2


def _gffn_body(eid_ref, ntu_ref, chg_ref, slot_ref, pref_ref,
               xs_ref, w1_hbm, w2_hbm, ys_ref, w1_v, w2_v,
               sem1a, sem2a, sem1b, sem2b):
    j = pl.program_id(0)
    s = slot_ref[j]
    nxt = pref_ref[j]

    @pl.when(chg_ref[j] == 1)
    def _():
        @pl.when(j == 0)
        def _():
            c1, c2 = _w_copies(w1_hbm, w2_hbm, w1_v, w2_v, sem1a, sem2a,
                               eid_ref[0], 0)
            c1.start()
            c2.start()

        @pl.when(s == 0)
        def _():
            c1, c2 = _w_copies(w1_hbm, w2_hbm, w1_v, w2_v, sem1a, sem2a,
                               eid_ref[j], 0)
            c1.wait()
            c2.wait()

        @pl.when(s == 1)
        def _():
            c1, c2 = _w_copies(w1_hbm, w2_hbm, w1_v, w2_v, sem1b, sem2b,
                               eid_ref[j], 1)
            c1.wait()
            c2.wait()

        @pl.when((nxt < E) & (s == 0))
        def _():
            c1, c2 = _w_copies(w1_hbm, w2_hbm, w1_v, w2_v, sem1b, sem2b,
                               nxt, 1)
            c1.start()
            c2.start()

        @pl.when((nxt < E) & (s == 1))
        def _():
            c1, c2 = _w_copies(w1_hbm, w2_hbm, w1_v, w2_v, sem1a, sem2a,
                               nxt, 0)
            c1.start()
            c2.start()

    def _compute(slot_static):
        xb = xs_ref[...].astype(jnp.bfloat16)
        hpre = lax.dot_general(
            xb, w1_v[slot_static], (((1,), (1,)), ((), ())),
            preferred_element_type=jnp.float32)
        hact = (hpre * 0.5 * (1.0 + lax.erf(hpre * 0.7071067811865476))
                ).astype(jnp.bfloat16)
        ys_ref[...] = lax.dot_general(
            hact, w2_v[slot_static], (((1,), (1,)), ((), ())),
            preferred_element_type=jnp.float32)

    @pl.when((j < ntu_ref[0]) & (s == 0))
    def _():
        _compute(0)

    @pl.when((j < ntu_ref[0]) & (s == 1))
    def _():
        _compute(1)


def _gffn(eid, ntu, chg, slot, pref, xs, W1b, W2b):
    grid_spec = pltpu.PrefetchScalarGridSpec(
        num_scalar_prefetch=5,
        grid=(NTS,),
        in_specs=[
            pl.BlockSpec((BTS, D), lambda j, *_: (j, 0)),
            pl.BlockSpec(memory_space=pl.ANY),
            pl.BlockSpec(memory_space=pl.ANY),
        ],
        out_specs=pl.BlockSpec((BTS, D), lambda j, *_: (j, 0)),
        scratch_shapes=[
            pltpu.VMEM((2, H, D), jnp.bfloat16),
            pltpu.VMEM((2, D, H), jnp.bfloat16),
            pltpu.SemaphoreType.DMA,
            pltpu.SemaphoreType.DMA,
            pltpu.SemaphoreType.DMA,
            pltpu.SemaphoreType.DMA,
        ],
    )
    return pl.pallas_call(
        _gffn_body,
        grid_spec=grid_spec,
        out_shape=jax.ShapeDtypeStruct((PADTOT, D), jnp.float32),
    )(eid, ntu, chg, slot, pref, xs, W1b, W2b)


def _combine_body(ys_hbm, p0_hbm, p1_hbm, g0_hbm, g1_hbm, out_hbm,
                  idx0_v, idx1_v, g0_v, g1_v, y0_v, y1_v, sem0, sem1):
    wid = lax.axis_index("s") * 2 + lax.axis_index("c")
    base = wid * TPW
    pltpu.sync_copy(p0_hbm.at[pl.ds(base, TPW)], idx0_v)
    pltpu.sync_copy(p1_hbm.at[pl.ds(base, TPW)], idx1_v)
    pltpu.sync_copy(g0_hbm.at[pl.ds(base, TPW)], g0_v)
    pltpu.sync_copy(g1_hbm.at[pl.ds(base, TPW)], g1_v)
    c0 = pltpu.async_copy(ys_hbm.at[idx0_v], y0_v, sem0)
    c1 = pltpu.async_copy(ys_hbm.at[idx1_v], y1_v, sem1)
    c0.wait()
    c1.wait()

    def _row(i, acc):
        g0 = g0_v[i]
        g1 = g1_v[i]
        for q in range(D // 16):
            sl = pl.ds(q * 16, 16)
            y0_v[i, sl] = y0_v[i, sl] * g0 + y1_v[i, sl] * g1
        return acc

    lax.fori_loop(0, TPW, _row, 0)
    pltpu.sync_copy(y0_v, out_hbm.at[pl.ds(base, TPW)])


def _sc_combine(ys, p0, p1, g0, g1):
    return pl.kernel(
        _combine_body,
        out_type=jax.ShapeDtypeStruct((T, D), jnp.float32),
        mesh=_sc_mesh(),
        scratch_types=[
            pltpu.VMEM((TPW,), jnp.int32),
            pltpu.VMEM((TPW,), jnp.int32),
            pltpu.VMEM((TPW, 16), jnp.float32),
            pltpu.VMEM((TPW, 16), jnp.float32),
            pltpu.VMEM((TPW, D), jnp.float32),
            pltpu.VMEM((TPW, D), jnp.float32),
            pltpu.SemaphoreType.DMA,
            pltpu.SemaphoreType.DMA,
        ],
    )(ys, p0, p1, g0, g1)


def kernel(x, Wr, W1, W2):
    b, t, d = x.shape
    x2 = x.reshape(t, d)
    g0, g1, p0, p1, eid, ntu, chg, slot, pref, lb = _router(x2, Wr)
    xs = _sc_dispatch(x2, p0, p1)
    ys = _gffn(eid, ntu, chg, slot, pref, xs,
               W1.astype(jnp.bfloat16), W2.astype(jnp.bfloat16))
    out = _sc_combine(ys, p0, p1, g0, g1)
    return out.reshape(b, t, d), lb.reshape(())


# speedup vs baseline: 1.3447x; 1.0116x over previous
import jax
import jax.numpy as jnp
from jax import lax
from jax.experimental import pallas as pl
from jax.experimental.pallas import tpu as pltpu
from jax.experimental.pallas import tpu_sc as plsc

T, D, H, E = 2048, 768, 3072, 8
BTS = 256
PADTOT = 4096 + E * BTS
NTS = PADTOT // BTS
CH = 512
NW = 32
TPW = T // NW


def _router_body(x_ref, wr_ref, g0_ref, g1_ref, p0_ref, p1_ref,
                 eid_ref, ntu_ref, chg_ref, slot_ref, pref_ref, lb_ref):
    x = x_ref[...]
    wr = wr_ref[...]
    logits = lax.dot_general(
        x, wr, (((1,), (1,)), ((), ())), preferred_element_type=jnp.float32)
    m = jnp.max(logits, axis=1, keepdims=True)
    ex = jnp.exp(logits - m)
    probs = ex / jnp.sum(ex, axis=1, keepdims=True)
    lane = lax.broadcasted_iota(jnp.int32, (T, E), 1)
    p1 = jnp.max(probs, axis=1, keepdims=True)
    a1 = jnp.min(jnp.where(probs >= p1, lane, E), axis=1, keepdims=True)
    m1 = lane == a1
    pm = jnp.where(m1, -1.0, probs)
    p2 = jnp.max(pm, axis=1, keepdims=True)
    a2 = jnp.min(jnp.where(pm >= p2, lane, E), axis=1, keepdims=True)
    m2 = lane == a2
    denom = p1 + p2 + 1e-9
    g0_ref[...] = jnp.broadcast_to(p1 / denom, (T, 16))
    g1_ref[...] = jnp.broadcast_to(p2 / denom, (T, 16))

    imp = jnp.sum(probs, axis=0, keepdims=True)
    load = jnp.sum(m1.astype(jnp.float32) + m2.astype(jnp.float32),
                   axis=0, keepdims=True)
    impn = imp / (jnp.sum(imp) + 1e-9)
    loadn = load / (jnp.sum(load) + 1e-9)
    lb_ref[...] = jnp.sum(impn * loadn, axis=1, keepdims=True) * E

    oh0 = m1.astype(jnp.float32)
    oh1 = m2.astype(jnp.float32)
    ri = lax.broadcasted_iota(jnp.int32, (CH, CH), 0)
    ci = lax.broadcasted_iota(jnp.int32, (CH, CH), 1)
    ls = (ci < ri).astype(jnp.float32)

    def _excl_cumsum(oh):
        carry = jnp.zeros((1, E), jnp.float32)
        parts = []
        for c in range(T // CH):
            blk = oh[c * CH:(c + 1) * CH, :]
            parts.append(lax.dot_general(
                ls, blk, (((1,), (0,)), ((), ())),
                preferred_element_type=jnp.float32) + carry)
            carry = carry + jnp.sum(blk, axis=0, keepdims=True)
        return jnp.concatenate(parts, axis=0), carry

    rank0, count0 = _excl_cumsum(oh0)
    rank1, count1 = _excl_cumsum(oh1)
    rank1 = rank1 + count0
    count = count0 + count1
    padded = jnp.floor((count + (BTS - 1)) * (1.0 / BTS)) * BTS
    er = lax.broadcasted_iota(jnp.int32, (E, E), 0)
    ec = lax.broadcasted_iota(jnp.int32, (E, E), 1)
    mstrict = (er < ec).astype(jnp.float32)
    offs = lax.dot_general(padded, mstrict, (((1,), (0,)), ((), ())),
                           preferred_element_type=jnp.float32)
    p0 = jnp.sum(jnp.where(m1, offs + rank0, 0.0), axis=1, keepdims=True)
    p1i = jnp.sum(jnp.where(m2, offs + rank1, 0.0), axis=1, keepdims=True)
    p0_ref[...] = p0.astype(jnp.int32).reshape(T)
    p1_ref[...] = p1i.astype(jnp.int32).reshape(T)

    ends = offs + padded
    jstart = (lax.broadcasted_iota(jnp.int32, (NTS, 1), 0) * BTS
              ).astype(jnp.float32)
    total = jnp.sum(padded, axis=1, keepdims=True)
    cnt = jnp.sum((jnp.broadcast_to(ends, (NTS, E)) <=
                   jnp.broadcast_to(jstart, (NTS, E))).astype(jnp.int32),
                  axis=1, keepdims=True)
    laste = jnp.sum((ends <= total - 1.0).astype(jnp.int32),
                    axis=1, keepdims=True)
    eid = jnp.where(jstart < total, cnt, laste)
    eid_ref[...] = eid.reshape(NTS)
    ntu_ref[...] = (total * (1.0 / BTS)).astype(jnp.int32).reshape(1)

    prev = jnp.concatenate(
        [jnp.full((1, 1), -1, jnp.int32), eid[:-1, :]], axis=0)
    chg = (eid != prev).astype(jnp.int32)
    ti = lax.broadcasted_iota(jnp.int32, (NTS, NTS), 0)
    tj = lax.broadcasted_iota(jnp.int32, (NTS, NTS), 1)
    lsi = (tj <= ti).astype(jnp.float32)
    csum = lax.dot_general(lsi, chg.astype(jnp.float32),
                           (((1,), (0,)), ((), ())),
                           preferred_element_type=jnp.float32)
    slot = lax.rem(csum.astype(jnp.int32) - 1, 2)
    lane2 = lax.broadcasted_iota(jnp.int32, (NTS, E), 1)
    present = jnp.broadcast_to(count > 0.0, (NTS, E))
    cand = jnp.where(present & (lane2 > jnp.broadcast_to(eid, (NTS, E))),
                     lane2, E)
    pref = jnp.min(cand, axis=1, keepdims=True)
    chg_ref[...] = chg.reshape(NTS)
    slot_ref[...] = slot.reshape(NTS)
    pref_ref[...] = pref.reshape(NTS)


def _router(x2, Wr):
    return pl.pallas_call(
        _router_body,
        out_shape=(
            jax.ShapeDtypeStruct((T, 16), jnp.float32),
            jax.ShapeDtypeStruct((T, 16), jnp.float32),
            jax.ShapeDtypeStruct((T,), jnp.int32),
            jax.ShapeDtypeStruct((T,), jnp.int32),
            jax.ShapeDtypeStruct((NTS,), jnp.int32),
            jax.ShapeDtypeStruct((1,), jnp.int32),
            jax.ShapeDtypeStruct((NTS,), jnp.int32),
            jax.ShapeDtypeStruct((NTS,), jnp.int32),
            jax.ShapeDtypeStruct((NTS,), jnp.int32),
            jax.ShapeDtypeStruct((1, 1), jnp.float32),
        ),
    )(x2, Wr)


def _sc_mesh():
    return plsc.VectorSubcoreMesh(core_axis_name="c", subcore_axis_name="s")


def _dispatch_body(x_hbm, p0_hbm, p1_hbm, xs_hbm, idx0_v, idx1_v, xbuf_v,
                   sem0, sem1, semx):
    wid = lax.axis_index("s") * 2 + lax.axis_index("c")
    base = wid * TPW
    cx = pltpu.async_copy(x_hbm.at[pl.ds(base, TPW)], xbuf_v, semx)
    pltpu.sync_copy(p0_hbm.at[pl.ds(base, TPW)], idx0_v)
    pltpu.sync_copy(p1_hbm.at[pl.ds(base, TPW)], idx1_v)
    cx.wait()
    c0 = pltpu.async_copy(xbuf_v, xs_hbm.at[idx0_v], sem0)
    c1 = pltpu.async_copy(xbuf_v, xs_hbm.at[idx1_v], sem1)
    c0.wait()
    c1.wait()


def _sc_dispatch(x2, p0, p1):
    return pl.kernel(
        _dispatch_body,
        out_type=jax.ShapeDtypeStruct((PADTOT, D), jnp.float32),
        mesh=_sc_mesh(),
        scratch_types=[
            pltpu.VMEM((TPW,), jnp.int32),
            pltpu.VMEM((TPW,), jnp.int32),
            pltpu.VMEM((TPW, D), jnp.float32),
            pltpu.SemaphoreType.DMA,
            pltpu.SemaphoreType.DMA,
            pltpu.SemaphoreType.DMA,
        ],
    )(x2, p0, p1)


def _w_copies(w1_hbm, w2_hbm, w1_v, w2_v, sem1, sem2, e, s):
    c1 = pltpu.make_async_copy(w1_hbm.at[e], w1_v.at[s], sem1)
    c2 = pltpu.make_async_copy(w2_hbm.at[e], w2_v.at[s], sem2)
    return c1, c2


def _gffn_body(eid_ref, ntu_ref, chg_ref, slot_ref, pref_ref,
               xs_ref, w1_hbm, w2_hbm, ys_ref, w1_v, w2_v,
               sem1a, sem2a, sem1b, sem2b):
    j = pl.program_id(0)
    s = slot_ref[j]
    nxt = pref_ref[j]

    @pl.when(chg_ref[j] == 1)
    def _():
        @pl.when(j == 0)
        def _():
            c1, c2 = _w_copies(w1_hbm, w2_hbm, w1_v, w2_v, sem1a, sem2a,
                               eid_ref[0], 0)
            c1.start()
            c2.start()

        @pl.when(s == 0)
        def _():
            c1, c2 = _w_copies(w1_hbm, w2_hbm, w1_v, w2_v, sem1a, sem2a,
                               eid_ref[j], 0)
            c1.wait()
            c2.wait()

        @pl.when(s == 1)
        def _():
            c1, c2 = _w_copies(w1_hbm, w2_hbm, w1_v, w2_v, sem1b, sem2b,
                               eid_ref[j], 1)
            c1.wait()
            c2.wait()

        @pl.when((nxt < E) & (s == 0))
        def _():
            c1, c2 = _w_copies(w1_hbm, w2_hbm, w1_v, w2_v, sem1b, sem2b,
                               nxt, 1)
            c1.start()
            c2.start()

        @pl.when((nxt < E) & (s == 1))
        def _():
            c1, c2 = _w_copies(w1_hbm, w2_hbm, w1_v, w2_v, sem1a, sem2a,
                               nxt, 0)
            c1.start()
            c2.start()

    def _compute(slot_static):
        xb = xs_ref[...].astype(jnp.bfloat16)
        hpre = lax.dot_general(
            xb, w1_v[slot_static], (((1,), (1,)), ((), ())),
            preferred_element_type=jnp.float32)
        hact = (hpre * 0.5 * (1.0 + lax.erf(hpre * 0.7071067811865476))
                ).astype(jnp.bfloat16)
        ys_ref[...] = lax.dot_general(
            hact, w2_v[slot_static], (((1,), (1,)), ((), ())),
            preferred_element_type=jnp.float32)

    @pl.when((j < ntu_ref[0]) & (s == 0))
    def _():
        _compute(0)

    @pl.when((j < ntu_ref[0]) & (s == 1))
    def _():
        _compute(1)


def _gffn(eid, ntu, chg, slot, pref, xs, W1b, W2b):
    grid_spec = pltpu.PrefetchScalarGridSpec(
        num_scalar_prefetch=5,
        grid=(NTS,),
        in_specs=[
            pl.BlockSpec((BTS, D), lambda j, *_: (j, 0)),
            pl.BlockSpec(memory_space=pl.ANY),
            pl.BlockSpec(memory_space=pl.ANY),
        ],
        out_specs=pl.BlockSpec((BTS, D), lambda j, *_: (j, 0)),
        scratch_shapes=[
            pltpu.VMEM((2, H, D), jnp.bfloat16),
            pltpu.VMEM((2, D, H), jnp.bfloat16),
            pltpu.SemaphoreType.DMA,
            pltpu.SemaphoreType.DMA,
            pltpu.SemaphoreType.DMA,
            pltpu.SemaphoreType.DMA,
        ],
    )
    return pl.pallas_call(
        _gffn_body,
        grid_spec=grid_spec,
        out_shape=jax.ShapeDtypeStruct((PADTOT, D), jnp.float32),
    )(eid, ntu, chg, slot, pref, xs, W1b, W2b)


def _combine_body(ys_hbm, p0_hbm, p1_hbm, g0_hbm, g1_hbm, out_hbm,
                  idx0_v, idx1_v, g0_v, g1_v, y0_v, y1_v, sem0, sem1):
    wid = lax.axis_index("s") * 2 + lax.axis_index("c")
    base = wid * TPW
    pltpu.sync_copy(p0_hbm.at[pl.ds(base, TPW)], idx0_v)
    pltpu.sync_copy(p1_hbm.at[pl.ds(base, TPW)], idx1_v)
    c0 = pltpu.async_copy(ys_hbm.at[idx0_v], y0_v, sem0)
    c1 = pltpu.async_copy(ys_hbm.at[idx1_v], y1_v, sem1)
    pltpu.sync_copy(g0_hbm.at[pl.ds(base, TPW)], g0_v)
    pltpu.sync_copy(g1_hbm.at[pl.ds(base, TPW)], g1_v)
    c0.wait()
    c1.wait()

    def _row(i, acc):
        g0 = g0_v[i]
        g1 = g1_v[i]
        for q in range(D // 16):
            sl = pl.ds(q * 16, 16)
            y0_v[i, sl] = y0_v[i, sl] * g0 + y1_v[i, sl] * g1
        return acc

    lax.fori_loop(0, TPW, _row, 0)
    pltpu.sync_copy(y0_v, out_hbm.at[pl.ds(base, TPW)])


def _sc_combine(ys, p0, p1, g0, g1):
    return pl.kernel(
        _combine_body,
        out_type=jax.ShapeDtypeStruct((T, D), jnp.float32),
        mesh=_sc_mesh(),
        scratch_types=[
            pltpu.VMEM((TPW,), jnp.int32),
            pltpu.VMEM((TPW,), jnp.int32),
            pltpu.VMEM((TPW, 16), jnp.float32),
            pltpu.VMEM((TPW, 16), jnp.float32),
            pltpu.VMEM((TPW, D), jnp.float32),
            pltpu.VMEM((TPW, D), jnp.float32),
            pltpu.SemaphoreType.DMA,
            pltpu.SemaphoreType.DMA,
        ],
    )(ys, p0, p1, g0, g1)


def kernel(x, Wr, W1, W2):
    b, t, d = x.shape
    x2 = x.reshape(t, d)
    g0, g1, p0, p1, eid, ntu, chg, slot, pref, lb = _router(x2, Wr)
    xs = _sc_dispatch(x2, p0, p1)
    ys = _gffn(eid, ntu, chg, slot, pref, xs,
               W1.astype(jnp.bfloat16), W2.astype(jnp.bfloat16))
    out = _sc_combine(ys, p0, p1, g0, g1)
    return out.reshape(b, t, d), lb.reshape(())
